# baseline (speedup 1.0000x reference)
import jax
import jax.numpy as jnp
from jax import lax
from jax.experimental import pallas as pl
from jax.experimental.pallas import tpu as pltpu
from jax.experimental.pallas import tpu_sc as plsc

_B, _Y, _X = 16, 4096, 512
_NEG = -10000
_L = 16
_UNROLL = 8


def _sc_hist(dur_hbm, olen_hbm, ilen_hbm, d_hbm,
             dbuf, sbuf, cbuf, obuf, lbuf, ibuf, sem):
    w = lax.axis_index("s")

    @pl.when(w < _B)
    def _():
        row = w
        in_cp = pltpu.async_copy(dur_hbm.at[row], dbuf.at[pl.ds(0, _Y)], sem)
        pltpu.sync_copy(olen_hbm, lbuf)
        pltpu.sync_copy(ilen_hbm, ibuf)

        lane = lax.iota(jnp.int32, _L)
        out_len = jnp.max(jnp.where(lane == row, lbuf[...], 0))
        max_in = jnp.max(ibuf[...])

        zeros = jnp.zeros((_L,), jnp.int32)

        def zero_s(j, carry):
            sbuf[pl.ds(j * _L, _L)] = zeros
            return carry

        lax.fori_loop(0, _X // _L, zero_s, 0)
        cbuf[pl.ds(0, _L)] = zeros
        in_cp.wait()

        lanep1 = lane + 1

        def step(base, is_final):
            val = dbuf[pl.ds(base, _L)]
            nxt = dbuf[pl.ds(base + 1, _L)]
            is_last = val != nxt
            if is_final:
                is_last = is_last | (lane == _L - 1)
            m1 = jnp.minimum(base + lanep1, out_len)
            plsc.store_scatter(sbuf, [val], m1, mask=is_last)

        def pass_row(i, carry):
            for u in range(_UNROLL):
                step((i * _UNROLL + u) * _L, False)
            return carry

        lax.fori_loop(0, _Y // (_L * _UNROLL) - 1, pass_row, 0)
        for u in range(_UNROLL - 1):
            step(_Y - _UNROLL * _L + u * _L, False)
        step(_Y - _L, True)

        def diff_o(j, run):
            base = j * _L
            s = sbuf[pl.ds(base, _L)]
            c = jnp.maximum(plsc.cummax(s), run)
            cbuf[pl.ds(base + _L, _L)] = c
            excl = plsc.load_gather(cbuf, [base + _L - 1 + lane])
            d = jnp.maximum(s - excl, 0)
            obuf[pl.ds(base, _L)] = jnp.where(base + lane < max_in, d, 0)
            return jnp.max(c)

        lax.fori_loop(0, _X // _L, diff_o, jnp.int32(0))

        pltpu.sync_copy(obuf, d_hbm.at[row])


def _tc_weights(dur_ref, olen_ref, out_ref):
    pos = lax.broadcasted_iota(jnp.int32, (_B, _Y), 1)
    mask = pos < olen_ref[...]
    dur = dur_ref[...]
    out_ref[...] = jnp.where(mask, dur, dur + _NEG)


@jax.jit
def kernel(duration, output_length, input_length):
    mesh = plsc.VectorSubcoreMesh(
        core_axis_name="c", subcore_axis_name="s", num_cores=1)
    hist = pl.kernel(
        _sc_hist,
        out_type=jax.ShapeDtypeStruct((_B, _X), jnp.int32),
        mesh=mesh,
        compiler_params=pltpu.CompilerParams(needs_layout_passes=False),
        scratch_types=[
            pltpu.VMEM((_Y + _L,), jnp.int32),
            pltpu.VMEM((_X,), jnp.int32),
            pltpu.VMEM((_X + _L,), jnp.int32),
            pltpu.VMEM((_X,), jnp.int32),
            pltpu.VMEM((_L,), jnp.int32),
            pltpu.VMEM((_L,), jnp.int32),
            pltpu.SemaphoreType.DMA,
        ],
    )
    durations = hist(duration, output_length, input_length)

    weights = pl.pallas_call(
        _tc_weights,
        out_shape=jax.ShapeDtypeStruct((_B, _Y), jnp.int32),
    )(duration, output_length.reshape(_B, 1))

    return (weights, durations)


# speedup vs baseline: 1.0727x; 1.0727x over previous
import jax
import jax.numpy as jnp
from jax import lax
from jax.experimental import pallas as pl
from jax.experimental.pallas import tpu as pltpu
from jax.experimental.pallas import tpu_sc as plsc

_B, _Y, _X = 16, 4096, 512
_NEG = -10000
_L = 16
_UNROLL = 8


def _sc_hist(dur_hbm, olen_hbm, ilen_hbm, d_hbm,
             dbuf, sbuf, cbuf, obuf, lbuf, ibuf, sem):
    w = lax.axis_index("s")

    @pl.when(w < _B)
    def _():
        row = w
        in_cp = pltpu.async_copy(dur_hbm.at[row], dbuf.at[pl.ds(0, _Y)], sem)
        pltpu.sync_copy(olen_hbm, lbuf)
        pltpu.sync_copy(ilen_hbm, ibuf)

        lane = lax.iota(jnp.int32, _L)
        out_len = jnp.max(jnp.where(lane == row, lbuf[...], 0))
        max_in = jnp.max(ibuf[...])

        zeros = jnp.zeros((_L,), jnp.int32)

        def zero_s(j, carry):
            sbuf[pl.ds(j * _L, _L)] = zeros
            return carry

        lax.fori_loop(0, _X // _L, zero_s, 0)
        cbuf[pl.ds(0, _L)] = zeros
        in_cp.wait()

        lanep1 = lane + 1

        def step(base, is_final):
            val = dbuf[pl.ds(base, _L)]
            nxt = dbuf[pl.ds(base + 1, _L)]
            is_last = val != nxt
            if is_final:
                is_last = is_last | (lane == _L - 1)
            m1 = jnp.minimum(base + lanep1, out_len)
            plsc.store_scatter(sbuf, [val], m1, mask=is_last)

        @plsc.parallel_loop(0, _Y // _L - 1, unroll=_UNROLL)
        def _(i):
            step(i * _L, False)

        step(_Y - _L, True)

        def diff_o(j, run):
            base = j * _L
            s = sbuf[pl.ds(base, _L)]
            c = jnp.maximum(plsc.cummax(s), run)
            cbuf[pl.ds(base + _L, _L)] = c
            excl = plsc.load_gather(cbuf, [base + _L - 1 + lane])
            d = jnp.maximum(s - excl, 0)
            obuf[pl.ds(base, _L)] = jnp.where(base + lane < max_in, d, 0)
            return jnp.max(c)

        lax.fori_loop(0, _X // _L, diff_o, jnp.int32(0))

        pltpu.sync_copy(obuf, d_hbm.at[row])


def _tc_weights(dur_ref, olen_ref, out_ref):
    pos = lax.broadcasted_iota(jnp.int32, (_B, _Y), 1)
    mask = pos < olen_ref[...]
    dur = dur_ref[...]
    out_ref[...] = jnp.where(mask, dur, dur + _NEG)


@jax.jit
def kernel(duration, output_length, input_length):
    mesh = plsc.VectorSubcoreMesh(
        core_axis_name="c", subcore_axis_name="s", num_cores=1)
    hist = pl.kernel(
        _sc_hist,
        out_type=jax.ShapeDtypeStruct((_B, _X), jnp.int32),
        mesh=mesh,
        compiler_params=pltpu.CompilerParams(needs_layout_passes=False),
        scratch_types=[
            pltpu.VMEM((_Y + _L,), jnp.int32),
            pltpu.VMEM((_X,), jnp.int32),
            pltpu.VMEM((_X + _L,), jnp.int32),
            pltpu.VMEM((_X,), jnp.int32),
            pltpu.VMEM((_L,), jnp.int32),
            pltpu.VMEM((_L,), jnp.int32),
            pltpu.SemaphoreType.DMA,
        ],
    )
    durations = hist(duration, output_length, input_length)

    weights = pl.pallas_call(
        _tc_weights,
        out_shape=jax.ShapeDtypeStruct((_B, _Y), jnp.int32),
    )(duration, output_length.reshape(_B, 1))

    return (weights, durations)


# speedup vs baseline: 1.0738x; 1.0010x over previous
import jax
import jax.numpy as jnp
from jax import lax
from jax.experimental import pallas as pl
from jax.experimental.pallas import tpu as pltpu
from jax.experimental.pallas import tpu_sc as plsc

_B, _Y, _X = 16, 4096, 512
_NEG = -10000
_L = 16
_UNROLL = 8


def _sc_hist(dur_hbm, olen_hbm, ilen_hbm, d_hbm,
             dbuf, sbuf, cbuf, obuf, lbuf, ibuf, sem, sem2):
    w = lax.axis_index("s")

    @pl.when(w < _B)
    def _():
        row = w
        in_cp1 = pltpu.async_copy(
            dur_hbm.at[row, pl.ds(0, _Y // 2)], dbuf.at[pl.ds(0, _Y // 2)],
            sem)
        in_cp2 = pltpu.async_copy(
            dur_hbm.at[row, pl.ds(_Y // 2, _Y // 2)],
            dbuf.at[pl.ds(_Y // 2, _Y // 2)], sem2)
        pltpu.sync_copy(olen_hbm, lbuf)
        pltpu.sync_copy(ilen_hbm, ibuf)

        lane = lax.iota(jnp.int32, _L)
        out_len = jnp.max(jnp.where(lane == row, lbuf[...], 0))
        max_in = jnp.max(ibuf[...])

        zeros = jnp.zeros((_L,), jnp.int32)

        def zero_s(j, carry):
            sbuf[pl.ds(j * _L, _L)] = zeros
            return carry

        lax.fori_loop(0, _X // _L, zero_s, 0)
        cbuf[pl.ds(0, _L)] = zeros

        lanep1 = lane + 1

        def step(base, is_final):
            val = dbuf[pl.ds(base, _L)]
            nxt = dbuf[pl.ds(base + 1, _L)]
            is_last = val != nxt
            if is_final:
                is_last = is_last | (lane == _L - 1)
            m1 = jnp.minimum(base + lanep1, out_len)
            plsc.store_scatter(sbuf, [val], m1, mask=is_last)

        in_cp1.wait()

        @plsc.parallel_loop(0, _Y // (2 * _L) - 1, unroll=_UNROLL)
        def _(i):
            step(i * _L, False)

        in_cp2.wait()

        @plsc.parallel_loop(_Y // (2 * _L) - 1, _Y // _L - 1,
                            unroll=_UNROLL)
        def _(i):
            step(i * _L, False)

        step(_Y - _L, True)

        def diff_o(j, run):
            base = j * _L
            s = sbuf[pl.ds(base, _L)]
            c = jnp.maximum(plsc.cummax(s), run)
            cbuf[pl.ds(base + _L, _L)] = c
            excl = plsc.load_gather(cbuf, [base + _L - 1 + lane])
            d = jnp.maximum(s - excl, 0)
            obuf[pl.ds(base, _L)] = jnp.where(base + lane < max_in, d, 0)
            return plsc.load_gather(cbuf, [zeros + (base + 2 * _L - 1)])

        lax.fori_loop(0, _X // _L, diff_o, zeros)

        pltpu.sync_copy(obuf, d_hbm.at[row])


def _tc_weights(dur_ref, olen_ref, out_ref):
    pos = lax.broadcasted_iota(jnp.int32, (_B, _Y), 1)
    mask = pos < olen_ref[...]
    dur = dur_ref[...]
    out_ref[...] = jnp.where(mask, dur, dur + _NEG)


@jax.jit
def kernel(duration, output_length, input_length):
    mesh = plsc.VectorSubcoreMesh(
        core_axis_name="c", subcore_axis_name="s", num_cores=1)
    hist = pl.kernel(
        _sc_hist,
        out_type=jax.ShapeDtypeStruct((_B, _X), jnp.int32),
        mesh=mesh,
        compiler_params=pltpu.CompilerParams(needs_layout_passes=False),
        scratch_types=[
            pltpu.VMEM((_Y + _L,), jnp.int32),
            pltpu.VMEM((_X,), jnp.int32),
            pltpu.VMEM((_X + _L,), jnp.int32),
            pltpu.VMEM((_X,), jnp.int32),
            pltpu.VMEM((_L,), jnp.int32),
            pltpu.VMEM((_L,), jnp.int32),
            pltpu.SemaphoreType.DMA,
            pltpu.SemaphoreType.DMA,
        ],
    )
    durations = hist(duration, output_length, input_length)

    weights = pl.pallas_call(
        _tc_weights,
        out_shape=jax.ShapeDtypeStruct((_B, _Y), jnp.int32),
    )(duration, output_length.reshape(_B, 1))

    return (weights, durations)
